# baseline (speedup 1.0000x reference)
import functools

import jax
import jax.numpy as jnp
from jax import lax
from jax.experimental import pallas as pl
from jax.experimental.pallas import tpu as pltpu
from jax.experimental.pallas import tpu_sc as plsc

N = 10000
E = 320000
D = 128
DF = 64
DS = D - DF
NP = 10240
NW = 32
C = 80
K = 128
B = 32
NB = C // B
EP = NW * C * K
RPT = NP // 16
KA = 64
CA = EP // (16 * KA)
BA = 16
NBA = CA // BA

@functools.cache
def _deg_kernel():
    mesh = plsc.VectorSubcoreMesh(core_axis_name="c", subcore_axis_name="s")
    return functools.partial(
        pl.kernel,
        mesh=mesh,
        out_type=jax.ShapeDtypeStruct((2 * NP,), jnp.float32),
        scratch_types=[
            pltpu.VMEM((C, K), jnp.int32),
            pltpu.VMEM((K,), jnp.float32),
            pltpu.VMEM((RPT,), jnp.float32),
            pltpu.VMEM_SHARED((NP,), jnp.float32),
        ],
    )(_deg_body)


def _deg_body(dst_hbm, out_hbm, dst_v, ones_v, stage_v, acc_sh):
    cid = lax.axis_index("c")
    sid = lax.axis_index("s")
    wid = sid * 2 + cid

    for i in range(K // 16):
        ones_v[pl.ds(i * 16, 16)] = jnp.ones((16,), jnp.float32)
    for i in range(RPT // 16):
        stage_v[pl.ds(i * 16, 16)] = jnp.zeros((16,), jnp.float32)

    pltpu.sync_copy(stage_v, acc_sh.at[pl.ds(sid * RPT, RPT)])
    plsc.subcore_barrier()

    pltpu.sync_copy(dst_hbm.at[pl.ds(wid * C, C)], dst_v)

    def body(c, carry):
        pltpu.sync_copy(ones_v, acc_sh.at[dst_v.at[c]], add=True)
        return carry

    lax.fori_loop(0, C, body, 0)
    plsc.subcore_barrier()

    pltpu.sync_copy(acc_sh.at[pl.ds(sid * RPT, RPT)], stage_v)
    pltpu.sync_copy(stage_v, out_hbm.at[pl.ds(cid * NP + sid * RPT, RPT)])


@functools.cache
def _agg_kernel():
    mesh = plsc.VectorSubcoreMesh(core_axis_name="c", subcore_axis_name="s")
    return functools.partial(
        pl.kernel,
        mesh=mesh,
        compiler_params=pltpu.CompilerParams(use_tc_tiling_on_sc=False),
        out_type=(
            jax.ShapeDtypeStruct((NP, DF), jnp.float32),
            jax.ShapeDtypeStruct((NP, DS), jnp.float32),
        ),
        scratch_types=[
            pltpu.VMEM((BA, KA), jnp.int32),
            pltpu.VMEM((BA, KA), jnp.int32),
            pltpu.VMEM((KA, DF), jnp.float32),
            pltpu.VMEM((KA, DF), jnp.float32),
            pltpu.VMEM((KA, DF), jnp.float32),
            pltpu.VMEM((KA, DF), jnp.float32),
            pltpu.VMEM((KA, DS), jnp.float32),
            pltpu.VMEM((KA, DS), jnp.float32),
            pltpu.VMEM((KA, DS), jnp.float32),
            pltpu.VMEM((KA, DS), jnp.float32),
            pltpu.VMEM_SHARED((NP, DF), jnp.float32),
            pltpu.VMEM_SHARED((NP, DS), jnp.float32),
            pltpu.SemaphoreType.DMA,
            pltpu.SemaphoreType.DMA,
            pltpu.SemaphoreType.DMA,
            pltpu.SemaphoreType.DMA,
            pltpu.SemaphoreType.DMA,
            pltpu.SemaphoreType.DMA,
            pltpu.SemaphoreType.DMA,
            pltpu.SemaphoreType.DMA,
            pltpu.SemaphoreType.DMA,
        ],
    )(_agg_body)


def _agg_body(hsa_hbm, hsb_hbm, src_hbm, dst_hbm, za_hbm, zb_hbm,
              outa_hbm, outb_hbm,
              sidx, didx, ra0, ra1, ra2, ra3, rb0, rb1, rb2, rb3,
              acca_sh, accb_sh,
              sg0, sg1, sg2, sg3, ss0, ss1, ss2, ss3, wsem):
    cid = lax.axis_index("c")
    sid = lax.axis_index("s")

    gsem = (sg0, sg1, sg2, sg3)
    ssem = (ss0, ss1, ss2, ss3)

    def init(z_hbm, rows, acc):
        pltpu.sync_copy(z_hbm, rows[0])
        for z in range(RPT // KA):
            pltpu.async_copy(
                rows[0], acc.at[pl.ds(sid * RPT + z * KA, KA)], wsem)
        for z in range(RPT // KA):
            pltpu.make_async_copy(
                rows[0], acc.at[pl.ds(sid * RPT + z * KA, KA)], wsem).wait()

    def edge_loop(hs_hbm, rows, acc):
        def start_gather(j, b):
            pltpu.async_copy(hs_hbm.at[sidx.at[j]], rows[b], gsem[b])

        def wait_gather(j, b):
            pltpu.make_async_copy(
                hs_hbm.at[sidx.at[j]], rows[b], gsem[b]).wait()

        def start_scatter(j, b):
            pltpu.async_copy(rows[b], acc.at[didx.at[j]], ssem[b], add=True)

        def wait_scatter(j, b):
            pltpu.make_async_copy(rows[b], acc.at[didx.at[j]], ssem[b]).wait()

        def blk_body(blk, carry):
            r0 = sid * CA + blk * BA
            pltpu.sync_copy(src_hbm.at[pl.ds(r0, BA)], sidx)
            pltpu.sync_copy(dst_hbm.at[pl.ds(r0, BA)], didx)
            for b in range(4):
                start_gather(b, b)

            def inner(i, c):
                for u in range(4):
                    jj = i * 4 + u
                    b = u
                    wait_gather(jj, b)
                    start_scatter(jj, b)

                    @pl.when((jj >= 1) & (jj + 3 < BA))
                    def _():
                        bp = (u - 1) % 4
                        wait_scatter(jj - 1, bp)
                        start_gather(jj + 3, bp)
                return c

            lax.fori_loop(0, BA // 4, inner, 0)
            for j in range(BA - 4, BA):
                wait_scatter(j, j % 4)
            return carry

        lax.fori_loop(0, NBA, blk_body, 0)

    def writeback(out_hbm, rows, acc):
        for z in range(RPT // KA):
            b = z % 2
            r0 = sid * RPT + z * KA
            if z >= 2:
                rp = sid * RPT + (z - 2) * KA
                pltpu.make_async_copy(
                    rows[b], out_hbm.at[pl.ds(rp, KA)], gsem[b]).wait()
            pltpu.sync_copy(acc.at[pl.ds(r0, KA)], rows[b])
            pltpu.async_copy(rows[b], out_hbm.at[pl.ds(r0, KA)], gsem[b])
        for z in range(RPT // KA - 2, RPT // KA):
            b = z % 2
            r0 = sid * RPT + z * KA
            pltpu.make_async_copy(
                rows[b], out_hbm.at[pl.ds(r0, KA)], gsem[b]).wait()

    @pl.when(cid == 0)
    def _():
        init(za_hbm, (ra0, ra1, ra2, ra3), acca_sh)

    @pl.when(cid == 1)
    def _():
        init(zb_hbm, (rb0, rb1, rb2, rb3), accb_sh)

    plsc.subcore_barrier()

    @pl.when(cid == 0)
    def _():
        edge_loop(hsa_hbm, (ra0, ra1, ra2, ra3), acca_sh)

    @pl.when(cid == 1)
    def _():
        edge_loop(hsb_hbm, (rb0, rb1, rb2, rb3), accb_sh)

    plsc.subcore_barrier()

    @pl.when(cid == 0)
    def _():
        writeback(outa_hbm, (ra0, ra1, ra2, ra3), acca_sh)

    @pl.when(cid == 1)
    def _():
        writeback(outb_hbm, (rb0, rb1, rb2, rb3), accb_sh)


_R = 2048


def _tc1_body(x_ref, w_ref, degp_ref, hsa_ref, hsb_ref, dinv_ref):
    deg = degp_ref[0, :] + degp_ref[1, :] + 1.0
    dinv = lax.rsqrt(jnp.maximum(deg, 1e-12))
    h = jnp.dot(x_ref[...], w_ref[...], preferred_element_type=jnp.float32)
    hs = h * dinv[:, None]
    hsa_ref[...] = hs[:, :DF]
    hsb_ref[...] = hs[:, DF:]
    dinv_ref[...] = dinv[:, None]


def _tc1(x_pad, W1, degp):
    return pl.pallas_call(
        _tc1_body,
        grid=(NP // _R,),
        in_specs=[
            pl.BlockSpec((_R, D), lambda i: (i, 0)),
            pl.BlockSpec((D, D), lambda i: (0, 0)),
            pl.BlockSpec((2, _R), lambda i: (0, i)),
        ],
        out_specs=[
            pl.BlockSpec((_R, DF), lambda i: (i, 0)),
            pl.BlockSpec((_R, DS), lambda i: (i, 0)),
            pl.BlockSpec((_R, 1), lambda i: (i, 0)),
        ],
        out_shape=[
            jax.ShapeDtypeStruct((NP, DF), jnp.float32),
            jax.ShapeDtypeStruct((NP, DS), jnp.float32),
            jax.ShapeDtypeStruct((NP, 1), jnp.float32),
        ],
    )(x_pad, W1, degp)


def _tc2_body(pa_ref, pb_ref, hsa_ref, hsb_ref, dinv_ref, b_ref, w_ref,
              outa_ref, outb_ref):
    agg = jnp.concatenate([pa_ref[...] + hsa_ref[...],
                           pb_ref[...] + hsb_ref[...]], axis=1)
    u = jnp.maximum(agg * dinv_ref[...] + b_ref[...], 0.0)
    h2 = jnp.dot(u, w_ref[...], preferred_element_type=jnp.float32)
    hs2 = h2 * dinv_ref[...]
    outa_ref[...] = hs2[:, :DF]
    outb_ref[...] = hs2[:, DF:]


def _tc2(pa, pb, hsa, hsb, dinv, b, W):
    return pl.pallas_call(
        _tc2_body,
        grid=(NP // _R,),
        in_specs=[
            pl.BlockSpec((_R, DF), lambda i: (i, 0)),
            pl.BlockSpec((_R, DS), lambda i: (i, 0)),
            pl.BlockSpec((_R, DF), lambda i: (i, 0)),
            pl.BlockSpec((_R, DS), lambda i: (i, 0)),
            pl.BlockSpec((_R, 1), lambda i: (i, 0)),
            pl.BlockSpec((1, D), lambda i: (0, 0)),
            pl.BlockSpec((D, D), lambda i: (0, 0)),
        ],
        out_specs=[
            pl.BlockSpec((_R, DF), lambda i: (i, 0)),
            pl.BlockSpec((_R, DS), lambda i: (i, 0)),
        ],
        out_shape=[
            jax.ShapeDtypeStruct((NP, DF), jnp.float32),
            jax.ShapeDtypeStruct((NP, DS), jnp.float32),
        ],
    )(pa, pb, hsa, hsb, dinv, b, W)


def _tc3_body(pa_ref, pb_ref, hsa_ref, hsb_ref, dinv_ref, b_ref, w_ref,
              blin_ref, out_ref):
    agg = jnp.concatenate([pa_ref[...] + hsa_ref[...],
                           pb_ref[...] + hsb_ref[...]], axis=1)
    u = jnp.maximum(agg * dinv_ref[...] + b_ref[...], 0.0)
    out_ref[...] = (
        jnp.dot(u, w_ref[...], preferred_element_type=jnp.float32)
        + blin_ref[...]
    )


def _tc3(pa, pb, hsa, hsb, dinv, b, W, blin):
    return pl.pallas_call(
        _tc3_body,
        grid=(NP // _R,),
        in_specs=[
            pl.BlockSpec((_R, DF), lambda i: (i, 0)),
            pl.BlockSpec((_R, DS), lambda i: (i, 0)),
            pl.BlockSpec((_R, DF), lambda i: (i, 0)),
            pl.BlockSpec((_R, DS), lambda i: (i, 0)),
            pl.BlockSpec((_R, 1), lambda i: (i, 0)),
            pl.BlockSpec((1, D), lambda i: (0, 0)),
            pl.BlockSpec((D, D), lambda i: (0, 0)),
            pl.BlockSpec((1, D), lambda i: (0, 0)),
        ],
        out_specs=pl.BlockSpec((_R, D), lambda i: (i, 0)),
        out_shape=jax.ShapeDtypeStruct((NP, D), jnp.float32),
    )(pa, pb, hsa, hsb, dinv, b, W, blin)


def kernel(x, edge_index, W1, b1, W2, b2, Wlin, blin):
    pad = jnp.full((EP - E,), N, dtype=jnp.int32)
    src_flat = jnp.concatenate([edge_index[0], pad])
    dst_flat = jnp.concatenate([edge_index[1], pad])
    dstp_deg = dst_flat.reshape(NW * C, K)
    srcp = src_flat.reshape(16 * CA, KA)
    dstp = dst_flat.reshape(16 * CA, KA)
    x_pad = jnp.zeros((NP, D), jnp.float32).at[:N].set(x)
    za = jnp.zeros((KA, DF), jnp.float32)
    zb = jnp.zeros((KA, DS), jnp.float32)

    degp = _deg_kernel()(dstp_deg).reshape(2, NP)
    hs1a, hs1b, dinv = _tc1(x_pad, W1, degp)
    p1a, p1b = _agg_kernel()(hs1a, hs1b, srcp, dstp, za, zb)
    hs2a, hs2b = _tc2(p1a, p1b, hs1a, hs1b, dinv, b1.reshape(1, D), W2)
    p2a, p2b = _agg_kernel()(hs2a, hs2b, srcp, dstp, za, zb)
    out = _tc3(p2a, p2b, hs2a, hs2b, dinv, b2.reshape(1, D), Wlin,
               blin.reshape(1, D))
    return out[:N]


# speedup vs baseline: 1.0720x; 1.0720x over previous
import functools

import jax
import jax.numpy as jnp
from jax import lax
from jax.experimental import pallas as pl
from jax.experimental.pallas import tpu as pltpu
from jax.experimental.pallas import tpu_sc as plsc

N = 10000
E = 320000
D = 128
DF = 64
DS = D - DF
NP = 10240
NW = 32
C = 80
K = 128
B = 32
NB = C // B
EP = NW * C * K
RPT = NP // 16
KA = 128
CA = EP // (16 * KA)
BA = 32
NBA = CA // BA

@functools.cache
def _deg_kernel():
    mesh = plsc.VectorSubcoreMesh(core_axis_name="c", subcore_axis_name="s")
    return functools.partial(
        pl.kernel,
        mesh=mesh,
        out_type=jax.ShapeDtypeStruct((2 * NP,), jnp.float32),
        scratch_types=[
            pltpu.VMEM((C, K), jnp.int32),
            pltpu.VMEM((K,), jnp.float32),
            pltpu.VMEM((RPT,), jnp.float32),
            pltpu.VMEM_SHARED((NP,), jnp.float32),
        ],
    )(_deg_body)


def _deg_body(dst_hbm, out_hbm, dst_v, ones_v, stage_v, acc_sh):
    cid = lax.axis_index("c")
    sid = lax.axis_index("s")
    wid = sid * 2 + cid

    for i in range(K // 16):
        ones_v[pl.ds(i * 16, 16)] = jnp.ones((16,), jnp.float32)
    for i in range(RPT // 16):
        stage_v[pl.ds(i * 16, 16)] = jnp.zeros((16,), jnp.float32)

    pltpu.sync_copy(stage_v, acc_sh.at[pl.ds(sid * RPT, RPT)])
    plsc.subcore_barrier()

    pltpu.sync_copy(dst_hbm.at[pl.ds(wid * C, C)], dst_v)

    def body(c, carry):
        pltpu.sync_copy(ones_v, acc_sh.at[dst_v.at[c]], add=True)
        return carry

    lax.fori_loop(0, C, body, 0)
    plsc.subcore_barrier()

    pltpu.sync_copy(acc_sh.at[pl.ds(sid * RPT, RPT)], stage_v)
    pltpu.sync_copy(stage_v, out_hbm.at[pl.ds(cid * NP + sid * RPT, RPT)])


@functools.cache
def _agg_kernel():
    mesh = plsc.VectorSubcoreMesh(core_axis_name="c", subcore_axis_name="s")
    return functools.partial(
        pl.kernel,
        mesh=mesh,
        compiler_params=pltpu.CompilerParams(use_tc_tiling_on_sc=False),
        out_type=(
            jax.ShapeDtypeStruct((NP, DF), jnp.float32),
            jax.ShapeDtypeStruct((NP, DS), jnp.float32),
        ),
        scratch_types=[
            pltpu.VMEM((BA, KA), jnp.int32),
            pltpu.VMEM((BA, KA), jnp.int32),
            pltpu.VMEM((KA, DF), jnp.float32),
            pltpu.VMEM((KA, DF), jnp.float32),
            pltpu.VMEM((KA, DS), jnp.float32),
            pltpu.VMEM((KA, DS), jnp.float32),
            pltpu.VMEM_SHARED((NP, DF), jnp.float32),
            pltpu.VMEM_SHARED((NP, DS), jnp.float32),
            pltpu.SemaphoreType.DMA,
            pltpu.SemaphoreType.DMA,
            pltpu.SemaphoreType.DMA,
            pltpu.SemaphoreType.DMA,
            pltpu.SemaphoreType.DMA,
        ],
    )(_agg_body)


def _agg_body(hsa_hbm, hsb_hbm, src_hbm, dst_hbm, za_hbm, zb_hbm,
              outa_hbm, outb_hbm,
              sidx, didx, ra0, ra1, rb0, rb1,
              acca_sh, accb_sh,
              sg0, sg1, ss0, ss1, wsem):
    cid = lax.axis_index("c")
    sid = lax.axis_index("s")

    gsem = (sg0, sg1)
    ssem = (ss0, ss1)

    def init(z_hbm, rows, acc):
        pltpu.sync_copy(z_hbm, rows[0])
        for z in range(RPT // KA):
            pltpu.async_copy(
                rows[0], acc.at[pl.ds(sid * RPT + z * KA, KA)], wsem)
        for z in range(RPT // KA):
            pltpu.make_async_copy(
                rows[0], acc.at[pl.ds(sid * RPT + z * KA, KA)], wsem).wait()

    def edge_loop(hs_hbm, rows, acc):
        def start_gather(j, b):
            pltpu.async_copy(hs_hbm.at[sidx.at[j]], rows[b], gsem[b])

        def wait_gather(j, b):
            pltpu.make_async_copy(
                hs_hbm.at[sidx.at[j]], rows[b], gsem[b]).wait()

        def start_scatter(j, b):
            pltpu.async_copy(rows[b], acc.at[didx.at[j]], ssem[b], add=True)

        def wait_scatter(j, b):
            pltpu.make_async_copy(rows[b], acc.at[didx.at[j]], ssem[b]).wait()

        def blk_body(blk, carry):
            r0 = sid * CA + blk * BA
            pltpu.sync_copy(src_hbm.at[pl.ds(r0, BA)], sidx)
            pltpu.sync_copy(dst_hbm.at[pl.ds(r0, BA)], didx)
            start_gather(0, 0)
            start_gather(1, 1)

            def inner(i, c):
                for b in range(2):
                    jj = i * 2 + b
                    wait_gather(jj, b)
                    start_scatter(jj, b)

                    @pl.when(jj + 2 < BA)
                    def _():
                        wait_scatter(jj, b)
                        start_gather(jj + 2, b)
                return c

            lax.fori_loop(0, BA // 2, inner, 0)
            wait_scatter(BA - 2, 0)
            wait_scatter(BA - 1, 1)
            return carry

        lax.fori_loop(0, NBA, blk_body, 0)

    def writeback(out_hbm, rows, acc):
        for z in range(RPT // KA):
            b = z % 2
            r0 = sid * RPT + z * KA
            if z >= 2:
                rp = sid * RPT + (z - 2) * KA
                pltpu.make_async_copy(
                    rows[b], out_hbm.at[pl.ds(rp, KA)], gsem[b]).wait()
            pltpu.sync_copy(acc.at[pl.ds(r0, KA)], rows[b])
            pltpu.async_copy(rows[b], out_hbm.at[pl.ds(r0, KA)], gsem[b])
        for z in range(RPT // KA - 2, RPT // KA):
            b = z % 2
            r0 = sid * RPT + z * KA
            pltpu.make_async_copy(
                rows[b], out_hbm.at[pl.ds(r0, KA)], gsem[b]).wait()

    @pl.when(cid == 0)
    def _():
        init(za_hbm, (ra0, ra1), acca_sh)

    @pl.when(cid == 1)
    def _():
        init(zb_hbm, (rb0, rb1), accb_sh)

    plsc.subcore_barrier()

    @pl.when(cid == 0)
    def _():
        edge_loop(hsa_hbm, (ra0, ra1), acca_sh)

    @pl.when(cid == 1)
    def _():
        edge_loop(hsb_hbm, (rb0, rb1), accb_sh)

    plsc.subcore_barrier()

    @pl.when(cid == 0)
    def _():
        writeback(outa_hbm, (ra0, ra1), acca_sh)

    @pl.when(cid == 1)
    def _():
        writeback(outb_hbm, (rb0, rb1), accb_sh)


_R = 2048


def _tc1_body(x_ref, w_ref, degp_ref, hsa_ref, hsb_ref, dinv_ref):
    deg = degp_ref[0, :] + degp_ref[1, :] + 1.0
    dinv = lax.rsqrt(jnp.maximum(deg, 1e-12))
    h = jnp.dot(x_ref[...], w_ref[...], preferred_element_type=jnp.float32)
    hs = h * dinv[:, None]
    hsa_ref[...] = hs[:, :DF]
    hsb_ref[...] = hs[:, DF:]
    dinv_ref[...] = dinv[:, None]


def _tc1(x_pad, W1, degp):
    return pl.pallas_call(
        _tc1_body,
        grid=(NP // _R,),
        in_specs=[
            pl.BlockSpec((_R, D), lambda i: (i, 0)),
            pl.BlockSpec((D, D), lambda i: (0, 0)),
            pl.BlockSpec((2, _R), lambda i: (0, i)),
        ],
        out_specs=[
            pl.BlockSpec((_R, DF), lambda i: (i, 0)),
            pl.BlockSpec((_R, DS), lambda i: (i, 0)),
            pl.BlockSpec((_R, 1), lambda i: (i, 0)),
        ],
        out_shape=[
            jax.ShapeDtypeStruct((NP, DF), jnp.float32),
            jax.ShapeDtypeStruct((NP, DS), jnp.float32),
            jax.ShapeDtypeStruct((NP, 1), jnp.float32),
        ],
    )(x_pad, W1, degp)


def _tc2_body(pa_ref, pb_ref, hsa_ref, hsb_ref, dinv_ref, b_ref, w_ref,
              outa_ref, outb_ref):
    agg = jnp.concatenate([pa_ref[...] + hsa_ref[...],
                           pb_ref[...] + hsb_ref[...]], axis=1)
    u = jnp.maximum(agg * dinv_ref[...] + b_ref[...], 0.0)
    h2 = jnp.dot(u, w_ref[...], preferred_element_type=jnp.float32)
    hs2 = h2 * dinv_ref[...]
    outa_ref[...] = hs2[:, :DF]
    outb_ref[...] = hs2[:, DF:]


def _tc2(pa, pb, hsa, hsb, dinv, b, W):
    return pl.pallas_call(
        _tc2_body,
        grid=(NP // _R,),
        in_specs=[
            pl.BlockSpec((_R, DF), lambda i: (i, 0)),
            pl.BlockSpec((_R, DS), lambda i: (i, 0)),
            pl.BlockSpec((_R, DF), lambda i: (i, 0)),
            pl.BlockSpec((_R, DS), lambda i: (i, 0)),
            pl.BlockSpec((_R, 1), lambda i: (i, 0)),
            pl.BlockSpec((1, D), lambda i: (0, 0)),
            pl.BlockSpec((D, D), lambda i: (0, 0)),
        ],
        out_specs=[
            pl.BlockSpec((_R, DF), lambda i: (i, 0)),
            pl.BlockSpec((_R, DS), lambda i: (i, 0)),
        ],
        out_shape=[
            jax.ShapeDtypeStruct((NP, DF), jnp.float32),
            jax.ShapeDtypeStruct((NP, DS), jnp.float32),
        ],
    )(pa, pb, hsa, hsb, dinv, b, W)


def _tc3_body(pa_ref, pb_ref, hsa_ref, hsb_ref, dinv_ref, b_ref, w_ref,
              blin_ref, out_ref):
    agg = jnp.concatenate([pa_ref[...] + hsa_ref[...],
                           pb_ref[...] + hsb_ref[...]], axis=1)
    u = jnp.maximum(agg * dinv_ref[...] + b_ref[...], 0.0)
    out_ref[...] = (
        jnp.dot(u, w_ref[...], preferred_element_type=jnp.float32)
        + blin_ref[...]
    )


def _tc3(pa, pb, hsa, hsb, dinv, b, W, blin):
    return pl.pallas_call(
        _tc3_body,
        grid=(NP // _R,),
        in_specs=[
            pl.BlockSpec((_R, DF), lambda i: (i, 0)),
            pl.BlockSpec((_R, DS), lambda i: (i, 0)),
            pl.BlockSpec((_R, DF), lambda i: (i, 0)),
            pl.BlockSpec((_R, DS), lambda i: (i, 0)),
            pl.BlockSpec((_R, 1), lambda i: (i, 0)),
            pl.BlockSpec((1, D), lambda i: (0, 0)),
            pl.BlockSpec((D, D), lambda i: (0, 0)),
            pl.BlockSpec((1, D), lambda i: (0, 0)),
        ],
        out_specs=pl.BlockSpec((_R, D), lambda i: (i, 0)),
        out_shape=jax.ShapeDtypeStruct((NP, D), jnp.float32),
    )(pa, pb, hsa, hsb, dinv, b, W, blin)


def kernel(x, edge_index, W1, b1, W2, b2, Wlin, blin):
    pad = jnp.full((EP - E,), N, dtype=jnp.int32)
    src_flat = jnp.concatenate([edge_index[0], pad])
    dst_flat = jnp.concatenate([edge_index[1], pad])
    dstp_deg = dst_flat.reshape(NW * C, K)
    srcp = src_flat.reshape(16 * CA, KA)
    dstp = dst_flat.reshape(16 * CA, KA)
    x_pad = jnp.zeros((NP, D), jnp.float32).at[:N].set(x)
    za = jnp.zeros((KA, DF), jnp.float32)
    zb = jnp.zeros((KA, DS), jnp.float32)

    degp = _deg_kernel()(dstp_deg).reshape(2, NP)
    hs1a, hs1b, dinv = _tc1(x_pad, W1, degp)
    p1a, p1b = _agg_kernel()(hs1a, hs1b, srcp, dstp, za, zb)
    hs2a, hs2b = _tc2(p1a, p1b, hs1a, hs1b, dinv, b1.reshape(1, D), W2)
    p2a, p2b = _agg_kernel()(hs2a, hs2b, srcp, dstp, za, zb)
    out = _tc3(p2a, p2b, hs2a, hs2b, dinv, b2.reshape(1, D), Wlin,
               blin.reshape(1, D))
    return out[:N]


# speedup vs baseline: 1.0758x; 1.0035x over previous
import functools

import jax
import jax.numpy as jnp
from jax import lax
from jax.experimental import pallas as pl
from jax.experimental.pallas import tpu as pltpu
from jax.experimental.pallas import tpu_sc as plsc

N = 10000
E = 320000
D = 128
DF = 64
DS = D - DF
NP = 10240
NW = 32
C = 80
K = 128
B = 32
NB = C // B
EP = NW * C * K
RPT = NP // 16
KA = 128
CA = EP // (16 * KA)
BA = 40
NBA = CA // BA

@functools.cache
def _deg_kernel():
    mesh = plsc.VectorSubcoreMesh(core_axis_name="c", subcore_axis_name="s")
    return functools.partial(
        pl.kernel,
        mesh=mesh,
        out_type=jax.ShapeDtypeStruct((2 * NP,), jnp.float32),
        scratch_types=[
            pltpu.VMEM((C, K), jnp.int32),
            pltpu.VMEM((K,), jnp.float32),
            pltpu.VMEM((RPT,), jnp.float32),
            pltpu.VMEM_SHARED((NP,), jnp.float32),
        ],
    )(_deg_body)


def _deg_body(dst_hbm, out_hbm, dst_v, ones_v, stage_v, acc_sh):
    cid = lax.axis_index("c")
    sid = lax.axis_index("s")
    wid = sid * 2 + cid

    for i in range(K // 16):
        ones_v[pl.ds(i * 16, 16)] = jnp.ones((16,), jnp.float32)
    for i in range(RPT // 16):
        stage_v[pl.ds(i * 16, 16)] = jnp.zeros((16,), jnp.float32)

    pltpu.sync_copy(stage_v, acc_sh.at[pl.ds(sid * RPT, RPT)])
    plsc.subcore_barrier()

    pltpu.sync_copy(dst_hbm.at[pl.ds(wid * C, C)], dst_v)

    def body(c, carry):
        pltpu.sync_copy(ones_v, acc_sh.at[dst_v.at[c]], add=True)
        return carry

    lax.fori_loop(0, C, body, 0)
    plsc.subcore_barrier()

    pltpu.sync_copy(acc_sh.at[pl.ds(sid * RPT, RPT)], stage_v)
    pltpu.sync_copy(stage_v, out_hbm.at[pl.ds(cid * NP + sid * RPT, RPT)])


@functools.cache
def _agg_kernel():
    mesh = plsc.VectorSubcoreMesh(core_axis_name="c", subcore_axis_name="s")
    return functools.partial(
        pl.kernel,
        mesh=mesh,
        compiler_params=pltpu.CompilerParams(use_tc_tiling_on_sc=False),
        out_type=(
            jax.ShapeDtypeStruct((NP, DF), jnp.float32),
            jax.ShapeDtypeStruct((NP, DS), jnp.float32),
        ),
        scratch_types=[
            pltpu.VMEM((BA, KA), jnp.int32),
            pltpu.VMEM((BA, KA), jnp.int32),
            pltpu.VMEM((KA, DF), jnp.float32),
            pltpu.VMEM((KA, DF), jnp.float32),
            pltpu.VMEM((KA, DS), jnp.float32),
            pltpu.VMEM((KA, DS), jnp.float32),
            pltpu.VMEM_SHARED((NP, DF), jnp.float32),
            pltpu.VMEM_SHARED((NP, DS), jnp.float32),
            pltpu.SemaphoreType.DMA,
            pltpu.SemaphoreType.DMA,
            pltpu.SemaphoreType.DMA,
            pltpu.SemaphoreType.DMA,
            pltpu.SemaphoreType.DMA,
        ],
    )(_agg_body)


def _agg_body(hsa_hbm, hsb_hbm, src_hbm, dst_hbm, za_hbm, zb_hbm,
              outa_hbm, outb_hbm,
              sidx, didx, ra0, ra1, rb0, rb1,
              acca_sh, accb_sh,
              sg0, sg1, ss0, ss1, wsem):
    cid = lax.axis_index("c")
    sid = lax.axis_index("s")

    gsem = (sg0, sg1)
    ssem = (ss0, ss1)

    def init(z_hbm, rows, acc):
        pltpu.sync_copy(z_hbm, rows[0])
        for z in range(RPT // KA):
            pltpu.async_copy(
                rows[0], acc.at[pl.ds(sid * RPT + z * KA, KA)], wsem)
        for z in range(RPT // KA):
            pltpu.make_async_copy(
                rows[0], acc.at[pl.ds(sid * RPT + z * KA, KA)], wsem).wait()

    def edge_loop(hs_hbm, rows, acc):
        def start_gather(j, b):
            pltpu.async_copy(hs_hbm.at[sidx.at[j]], rows[b], gsem[b])

        def wait_gather(j, b):
            pltpu.make_async_copy(
                hs_hbm.at[sidx.at[j]], rows[b], gsem[b]).wait()

        def start_scatter(j, b):
            pltpu.async_copy(rows[b], acc.at[didx.at[j]], ssem[b], add=True)

        def wait_scatter(j, b):
            pltpu.make_async_copy(rows[b], acc.at[didx.at[j]], ssem[b]).wait()

        def blk_body(blk, carry):
            r0 = sid * CA + blk * BA
            pltpu.sync_copy(src_hbm.at[pl.ds(r0, BA)], sidx)
            pltpu.sync_copy(dst_hbm.at[pl.ds(r0, BA)], didx)
            start_gather(0, 0)
            start_gather(1, 1)

            def inner(i, c):
                for b in range(2):
                    jj = i * 2 + b
                    wait_gather(jj, b)
                    start_scatter(jj, b)

                    @pl.when(jj + 2 < BA)
                    def _():
                        wait_scatter(jj, b)
                        start_gather(jj + 2, b)
                return c

            lax.fori_loop(0, BA // 2, inner, 0)
            wait_scatter(BA - 2, 0)
            wait_scatter(BA - 1, 1)
            return carry

        lax.fori_loop(0, NBA, blk_body, 0)

    def writeback(out_hbm, rows, acc):
        for z in range(RPT // KA):
            b = z % 2
            r0 = sid * RPT + z * KA
            if z >= 2:
                rp = sid * RPT + (z - 2) * KA
                pltpu.make_async_copy(
                    rows[b], out_hbm.at[pl.ds(rp, KA)], gsem[b]).wait()
            pltpu.sync_copy(acc.at[pl.ds(r0, KA)], rows[b])
            pltpu.async_copy(rows[b], out_hbm.at[pl.ds(r0, KA)], gsem[b])
        for z in range(RPT // KA - 2, RPT // KA):
            b = z % 2
            r0 = sid * RPT + z * KA
            pltpu.make_async_copy(
                rows[b], out_hbm.at[pl.ds(r0, KA)], gsem[b]).wait()

    @pl.when(cid == 0)
    def _():
        init(za_hbm, (ra0, ra1), acca_sh)

    @pl.when(cid == 1)
    def _():
        init(zb_hbm, (rb0, rb1), accb_sh)

    plsc.subcore_barrier()

    @pl.when(cid == 0)
    def _():
        edge_loop(hsa_hbm, (ra0, ra1), acca_sh)

    @pl.when(cid == 1)
    def _():
        edge_loop(hsb_hbm, (rb0, rb1), accb_sh)

    plsc.subcore_barrier()

    @pl.when(cid == 0)
    def _():
        writeback(outa_hbm, (ra0, ra1), acca_sh)

    @pl.when(cid == 1)
    def _():
        writeback(outb_hbm, (rb0, rb1), accb_sh)


_R = 2048


def _tc1_body(x_ref, w_ref, degp_ref, hsa_ref, hsb_ref, dinv_ref):
    deg = degp_ref[0, :] + degp_ref[1, :] + 1.0
    dinv = lax.rsqrt(jnp.maximum(deg, 1e-12))
    h = jnp.dot(x_ref[...], w_ref[...], preferred_element_type=jnp.float32)
    hs = h * dinv[:, None]
    hsa_ref[...] = hs[:, :DF]
    hsb_ref[...] = hs[:, DF:]
    dinv_ref[...] = dinv[:, None]


def _tc1(x_pad, W1, degp):
    return pl.pallas_call(
        _tc1_body,
        grid=(NP // _R,),
        in_specs=[
            pl.BlockSpec((_R, D), lambda i: (i, 0)),
            pl.BlockSpec((D, D), lambda i: (0, 0)),
            pl.BlockSpec((2, _R), lambda i: (0, i)),
        ],
        out_specs=[
            pl.BlockSpec((_R, DF), lambda i: (i, 0)),
            pl.BlockSpec((_R, DS), lambda i: (i, 0)),
            pl.BlockSpec((_R, 1), lambda i: (i, 0)),
        ],
        out_shape=[
            jax.ShapeDtypeStruct((NP, DF), jnp.float32),
            jax.ShapeDtypeStruct((NP, DS), jnp.float32),
            jax.ShapeDtypeStruct((NP, 1), jnp.float32),
        ],
    )(x_pad, W1, degp)


def _tc2_body(pa_ref, pb_ref, hsa_ref, hsb_ref, dinv_ref, b_ref, w_ref,
              outa_ref, outb_ref):
    agg = jnp.concatenate([pa_ref[...] + hsa_ref[...],
                           pb_ref[...] + hsb_ref[...]], axis=1)
    u = jnp.maximum(agg * dinv_ref[...] + b_ref[...], 0.0)
    h2 = jnp.dot(u, w_ref[...], preferred_element_type=jnp.float32)
    hs2 = h2 * dinv_ref[...]
    outa_ref[...] = hs2[:, :DF]
    outb_ref[...] = hs2[:, DF:]


def _tc2(pa, pb, hsa, hsb, dinv, b, W):
    return pl.pallas_call(
        _tc2_body,
        grid=(NP // _R,),
        in_specs=[
            pl.BlockSpec((_R, DF), lambda i: (i, 0)),
            pl.BlockSpec((_R, DS), lambda i: (i, 0)),
            pl.BlockSpec((_R, DF), lambda i: (i, 0)),
            pl.BlockSpec((_R, DS), lambda i: (i, 0)),
            pl.BlockSpec((_R, 1), lambda i: (i, 0)),
            pl.BlockSpec((1, D), lambda i: (0, 0)),
            pl.BlockSpec((D, D), lambda i: (0, 0)),
        ],
        out_specs=[
            pl.BlockSpec((_R, DF), lambda i: (i, 0)),
            pl.BlockSpec((_R, DS), lambda i: (i, 0)),
        ],
        out_shape=[
            jax.ShapeDtypeStruct((NP, DF), jnp.float32),
            jax.ShapeDtypeStruct((NP, DS), jnp.float32),
        ],
    )(pa, pb, hsa, hsb, dinv, b, W)


def _tc3_body(pa_ref, pb_ref, hsa_ref, hsb_ref, dinv_ref, b_ref, w_ref,
              blin_ref, out_ref):
    agg = jnp.concatenate([pa_ref[...] + hsa_ref[...],
                           pb_ref[...] + hsb_ref[...]], axis=1)
    u = jnp.maximum(agg * dinv_ref[...] + b_ref[...], 0.0)
    out_ref[...] = (
        jnp.dot(u, w_ref[...], preferred_element_type=jnp.float32)
        + blin_ref[...]
    )


def _tc3(pa, pb, hsa, hsb, dinv, b, W, blin):
    return pl.pallas_call(
        _tc3_body,
        grid=(NP // _R,),
        in_specs=[
            pl.BlockSpec((_R, DF), lambda i: (i, 0)),
            pl.BlockSpec((_R, DS), lambda i: (i, 0)),
            pl.BlockSpec((_R, DF), lambda i: (i, 0)),
            pl.BlockSpec((_R, DS), lambda i: (i, 0)),
            pl.BlockSpec((_R, 1), lambda i: (i, 0)),
            pl.BlockSpec((1, D), lambda i: (0, 0)),
            pl.BlockSpec((D, D), lambda i: (0, 0)),
            pl.BlockSpec((1, D), lambda i: (0, 0)),
        ],
        out_specs=pl.BlockSpec((_R, D), lambda i: (i, 0)),
        out_shape=jax.ShapeDtypeStruct((NP, D), jnp.float32),
    )(pa, pb, hsa, hsb, dinv, b, W, blin)


def kernel(x, edge_index, W1, b1, W2, b2, Wlin, blin):
    pad = jnp.full((EP - E,), N, dtype=jnp.int32)
    src_flat = jnp.concatenate([edge_index[0], pad])
    dst_flat = jnp.concatenate([edge_index[1], pad])
    dstp_deg = dst_flat.reshape(NW * C, K)
    srcp = src_flat.reshape(16 * CA, KA)
    dstp = dst_flat.reshape(16 * CA, KA)
    x_pad = jnp.zeros((NP, D), jnp.float32).at[:N].set(x)
    za = jnp.zeros((KA, DF), jnp.float32)
    zb = jnp.zeros((KA, DS), jnp.float32)

    degp = _deg_kernel()(dstp_deg).reshape(2, NP)
    hs1a, hs1b, dinv = _tc1(x_pad, W1, degp)
    p1a, p1b = _agg_kernel()(hs1a, hs1b, srcp, dstp, za, zb)
    hs2a, hs2b = _tc2(p1a, p1b, hs1a, hs1b, dinv, b1.reshape(1, D), W2)
    p2a, p2b = _agg_kernel()(hs2a, hs2b, srcp, dstp, za, zb)
    out = _tc3(p2a, p2b, hs2a, hs2b, dinv, b2.reshape(1, D), Wlin,
               blin.reshape(1, D))
    return out[:N]
